# baseline (speedup 1.0000x reference)
import jax
import jax.numpy as jnp
from jax.experimental import pallas as pl

_B_TILE = 32768
_K = 12


def _tc_body(x_ref, w_ref, o_ref):
    x = x_ref[...]
    w = w_ref[...]
    s = jax.lax.dot_general(w, x, (((1,), (1,)), ((), ())),
                            preferred_element_type=jnp.float32)
    a = jnp.abs(s)
    tb = a.shape[1]
    bits = jax.lax.bitcast_convert_type(a, jnp.int32)
    sub = jax.lax.broadcasted_iota(jnp.int32, a.shape, 0)
    cur = jax.lax.bitcast_convert_type(
        jnp.bitwise_or(jnp.bitwise_and(bits, ~jnp.int32(31)), sub),
        jnp.float32)
    t = [cur[0:8], cur[8:16], cur[16:24], cur[24:32]]

    def _ce(i, j):
        hi = jnp.maximum(t[i], t[j])
        lo = jnp.minimum(t[i], t[j])
        t[i], t[j] = hi, lo

    _ce(0, 1); _ce(2, 3); _ce(0, 2); _ce(1, 3); _ce(1, 2)
    acc = jnp.zeros((1, tb), jnp.float32)
    for _ in range(_K):
        m = jnp.max(t[0], axis=0, keepdims=True)
        acc = acc + m
        eq = t[0] == m
        t[0] = jnp.where(eq, t[1], t[0])
        t[1] = jnp.where(eq, t[2], t[1])
        t[2] = jnp.where(eq, t[3], t[2])
        t[3] = jnp.where(eq, -0.0, t[3])
    o_ref[...] = acc * (1.0 / _K)


def kernel(embedding, W):
    B, emb = embedding.shape
    out = pl.pallas_call(
        _tc_body,
        grid=(B // _B_TILE,),
        in_specs=[
            pl.BlockSpec((_B_TILE, emb), lambda i: (i, 0)),
            pl.BlockSpec(W.shape, lambda i: (0, 0)),
        ],
        out_specs=pl.BlockSpec((1, _B_TILE), lambda i: (0, i)),
        out_shape=jax.ShapeDtypeStruct((1, B), jnp.float32),
    )(embedding, W)
    return out.reshape(B, 1)


# speedup vs baseline: 1.0304x; 1.0304x over previous
import jax
import jax.numpy as jnp
from jax.experimental import pallas as pl

_B_TILE = 16384
_K = 12


def _tc_body(x_ref, w_ref, o_ref):
    x = x_ref[...]
    w = w_ref[...]
    s = jax.lax.dot_general(w, x, (((1,), (1,)), ((), ())),
                            preferred_element_type=jnp.float32)
    a = jnp.abs(s)
    tb = a.shape[1]
    bits = jax.lax.bitcast_convert_type(a, jnp.int32)
    sub = jax.lax.broadcasted_iota(jnp.int32, a.shape, 0)
    cur = jax.lax.bitcast_convert_type(
        jnp.bitwise_or(jnp.bitwise_and(bits, ~jnp.int32(31)), sub),
        jnp.float32)
    t = [cur[0:8], cur[8:16], cur[16:24], cur[24:32]]

    def _ce(i, j):
        hi = jnp.maximum(t[i], t[j])
        lo = jnp.minimum(t[i], t[j])
        t[i], t[j] = hi, lo

    _ce(0, 1); _ce(2, 3); _ce(0, 2); _ce(1, 3); _ce(1, 2)
    acc = jnp.zeros((1, tb), jnp.float32)
    for _ in range(_K):
        m = jnp.max(t[0], axis=0, keepdims=True)
        acc = acc + m
        eq = t[0] == m
        t[0] = jnp.where(eq, t[1], t[0])
        t[1] = jnp.where(eq, t[2], t[1])
        t[2] = jnp.where(eq, t[3], t[2])
        t[3] = jnp.where(eq, -0.0, t[3])
    o_ref[...] = acc * (1.0 / _K)


def kernel(embedding, W):
    B, emb = embedding.shape
    out = pl.pallas_call(
        _tc_body,
        grid=(B // _B_TILE,),
        in_specs=[
            pl.BlockSpec((_B_TILE, emb), lambda i: (i, 0)),
            pl.BlockSpec(W.shape, lambda i: (0, 0)),
        ],
        out_specs=pl.BlockSpec((1, _B_TILE), lambda i: (0, i)),
        out_shape=jax.ShapeDtypeStruct((1, B), jnp.float32),
    )(embedding, W)
    return out.reshape(B, 1)


# speedup vs baseline: 1.0435x; 1.0126x over previous
import jax
import jax.numpy as jnp
from jax.experimental import pallas as pl
from jax.experimental.pallas import tpu as pltpu

_B_TILE = 16384
_K = 12


def _tc_body(x_ref, w_ref, o_ref):
    x = x_ref[...]
    w = w_ref[...]
    s = jax.lax.dot_general(w, x, (((1,), (1,)), ((), ())),
                            preferred_element_type=jnp.float32)
    tb = s.shape[1]
    bits = jax.lax.bitcast_convert_type(s, jnp.int32)
    sub = jax.lax.broadcasted_iota(jnp.int32, s.shape, 0)
    cur = jax.lax.bitcast_convert_type(
        jnp.bitwise_or(jnp.bitwise_and(bits, jnp.int32(0x7FFFFFE0)), sub),
        jnp.float32)
    t = [cur[0:8], cur[8:16], cur[16:24], cur[24:32]]

    def _ce(i, j):
        hi = jnp.maximum(t[i], t[j])
        lo = jnp.minimum(t[i], t[j])
        t[i], t[j] = hi, lo

    _ce(0, 1); _ce(2, 3); _ce(0, 2); _ce(1, 3); _ce(1, 2)
    acc = jnp.zeros((1, tb), jnp.float32)
    for _ in range(_K):
        m = t[0]
        for d in (1, 2, 4):
            m = jnp.maximum(m, pltpu.roll(m, d, 0))
        acc = acc + m[0:1]
        eq = t[0] == m
        t[0] = jnp.where(eq, t[1], t[0])
        t[1] = jnp.where(eq, t[2], t[1])
        t[2] = jnp.where(eq, t[3], t[2])
        t[3] = jnp.where(eq, -0.0, t[3])
    o_ref[...] = acc * (1.0 / _K)


def kernel(embedding, W):
    B, emb = embedding.shape
    out = pl.pallas_call(
        _tc_body,
        grid=(B // _B_TILE,),
        in_specs=[
            pl.BlockSpec((_B_TILE, emb), lambda i: (i, 0)),
            pl.BlockSpec(W.shape, lambda i: (0, 0)),
        ],
        out_specs=pl.BlockSpec((1, _B_TILE), lambda i: (0, i)),
        out_shape=jax.ShapeDtypeStruct((1, B), jnp.float32),
    )(embedding, W)
    return out.reshape(B, 1)
